# baseline (device time: 181164 ns/iter reference)
import jax
import jax.numpy as jnp
from jax import lax
from jax.experimental import pallas as pl
from jax.experimental.pallas import tpu as pltpu

B, SQ, H, D = 8, 8, 16, 128
SCALE = D ** -0.5


def _local_body(r_ref, q_ref, k_ref, v_ref, o_ref, m_ref, l_ref):
    del r_ref
    h = pl.program_id(0)
    q = q_ref[0].astype(jnp.bfloat16)
    k = k_ref[0].astype(jnp.bfloat16)
    s = lax.dot_general(
        q, k, (((1,), (1,)), ((), ())),
        preferred_element_type=jnp.float32) * SCALE
    m = jnp.max(s, axis=1, keepdims=True)
    p = jnp.exp(s - m)
    l = jnp.sum(p, axis=1, keepdims=True)
    v = v_ref[0].astype(jnp.bfloat16)
    o = lax.dot_general(
        p.astype(jnp.bfloat16), v, (((1,), (0,)), ((), ())),
        preferred_element_type=jnp.float32)
    o_ref[0, :, pl.ds(h * D, D)] = o
    m_ref[pl.ds(h, 1)] = m[None]
    l_ref[pl.ds(h, 1)] = l[None]


def _local_partial(Q, K, V, rid):
    skv = K.shape[1]
    Qr = Q.reshape(B, SQ, H * D)
    Kr = K.reshape(B, skv, H * D)
    Vr = V.reshape(B, skv, H * D)
    grid_spec = pltpu.PrefetchScalarGridSpec(
        num_scalar_prefetch=1,
        grid=(H,),
        in_specs=[
            pl.BlockSpec((1, SQ, D), lambda h, r: (r[0], 0, h)),
            pl.BlockSpec((1, skv, D), lambda h, r: (r[0], 0, h)),
            pl.BlockSpec((1, skv, D), lambda h, r: (r[0], 0, h)),
        ],
        out_specs=[
            pl.BlockSpec((1, SQ, H * D), lambda h, r: (0, 0, 0)),
            pl.BlockSpec((H, SQ, 1), lambda h, r: (0, 0, 0)),
            pl.BlockSpec((H, SQ, 1), lambda h, r: (0, 0, 0)),
        ],
    )
    o_part, m_part, l_part = pl.pallas_call(
        _local_body,
        grid_spec=grid_spec,
        out_shape=[
            jax.ShapeDtypeStruct((1, SQ, H * D), jnp.float32),
            jax.ShapeDtypeStruct((H, SQ, 1), jnp.float32),
            jax.ShapeDtypeStruct((H, SQ, 1), jnp.float32),
        ],
    )(rid, Qr, Kr, Vr)
    return (o_part.reshape(1, SQ, H, D),
            jnp.transpose(m_part, (2, 1, 0)),
            jnp.transpose(l_part, (2, 1, 0)))


def _combine_body(o_ref, m_ref, l_ref, out_ref,
                  acc_o, ml_acc, zrecv_o, zrecv_ml,
                  z_send_o_sem, z_recv_o_sem, z_send_ml_sem, z_recv_ml_sem,
                  xy_send_sem, xy_recv_sem):
    mx = lax.axis_index("x")
    my = lax.axis_index("y")
    mz = lax.axis_index("z")
    rid = mx * 4 + my

    partners = [
        (mx, my, mz ^ 1),
        (mx, my, mz ^ 2),
        (mx, my ^ 1, mz),
        (mx, my ^ 2, mz),
        (mx ^ 1, my, mz),
    ]
    barrier = pltpu.get_barrier_semaphore()
    for dev in partners:
        pl.semaphore_signal(barrier, inc=1, device_id=dev,
                            device_id_type=pl.DeviceIdType.MESH)
    pl.semaphore_wait(barrier, len(partners))

    acc_o[...] = o_ref[...]
    m_acc = m_ref[...]
    l_acc = l_ref[...]

    for s in range(2):
        ml_acc[0] = m_acc
        ml_acc[1] = l_acc
        dev = (mx, my, mz ^ (1 << s))
        rdma_o = pltpu.make_async_remote_copy(
            src_ref=acc_o, dst_ref=zrecv_o.at[s],
            send_sem=z_send_o_sem.at[s], recv_sem=z_recv_o_sem.at[s],
            device_id=dev, device_id_type=pl.DeviceIdType.MESH)
        rdma_ml = pltpu.make_async_remote_copy(
            src_ref=ml_acc, dst_ref=zrecv_ml.at[s],
            send_sem=z_send_ml_sem.at[s], recv_sem=z_recv_ml_sem.at[s],
            device_id=dev, device_id_type=pl.DeviceIdType.MESH)
        rdma_o.start()
        rdma_ml.start()
        rdma_ml.wait()
        rdma_o.wait()

        m_other = zrecv_ml[s, 0]
        l_other = zrecv_ml[s, 1]
        m_new = jnp.maximum(m_acc, m_other)
        a_self = jnp.exp(m_acc - m_new)
        a_other = jnp.exp(m_other - m_new)
        acc_o[...] = (acc_o[...] * a_self[..., None]
                      + zrecv_o[s] * a_other[..., None])
        l_acc = l_acc * a_self + l_other * a_other
        m_acc = m_new

    out_ref[pl.ds(rid, 1)] = acc_o[...] / l_acc[..., None]

    xy_partners = [
        (mx, my ^ 1, mz),
        (mx, my ^ 2, mz),
        (mx ^ 1, my, mz),
    ]
    for k in range(3):
        size = 1 << k
        base = (rid // size) * size
        rdma = pltpu.make_async_remote_copy(
            src_ref=out_ref.at[pl.ds(base, size)],
            dst_ref=out_ref.at[pl.ds(base, size)],
            send_sem=xy_send_sem.at[k], recv_sem=xy_recv_sem.at[k],
            device_id=xy_partners[k], device_id_type=pl.DeviceIdType.MESH)
        rdma.start()
        rdma.wait()


def _combine(o_part, m_part, l_part):
    return pl.pallas_call(
        _combine_body,
        in_specs=[
            pl.BlockSpec(memory_space=pltpu.VMEM),
            pl.BlockSpec(memory_space=pltpu.VMEM),
            pl.BlockSpec(memory_space=pltpu.VMEM),
        ],
        out_specs=pl.BlockSpec(memory_space=pltpu.VMEM),
        out_shape=jax.ShapeDtypeStruct((B, SQ, H, D), jnp.float32),
        scratch_shapes=[
            pltpu.VMEM((1, SQ, H, D), jnp.float32),
            pltpu.VMEM((2, 1, SQ, H), jnp.float32),
            pltpu.VMEM((2, 1, SQ, H, D), jnp.float32),
            pltpu.VMEM((2, 2, 1, SQ, H), jnp.float32),
            pltpu.SemaphoreType.DMA((2,)),
            pltpu.SemaphoreType.DMA((2,)),
            pltpu.SemaphoreType.DMA((2,)),
            pltpu.SemaphoreType.DMA((2,)),
            pltpu.SemaphoreType.DMA((3,)),
            pltpu.SemaphoreType.DMA((3,)),
        ],
        compiler_params=pltpu.CompilerParams(collective_id=0),
    )(o_part, m_part, l_part)


def kernel(Q, K, V):
    rid = (lax.axis_index("x") * 4 + lax.axis_index("y")).astype(jnp.int32)
    rid_arr = jnp.reshape(rid, (1,))
    o_part, m_part, l_part = _local_partial(Q, K, V, rid_arr)
    return _combine(o_part, m_part, l_part)


# device time: 44248 ns/iter; 4.0943x vs baseline; 4.0943x over previous
import jax
import jax.numpy as jnp
from jax import lax
from jax.experimental import pallas as pl
from jax.experimental.pallas import tpu as pltpu

B, SQ, H, D = 8, 8, 16, 128
SKV = 1024
SCALE = D ** -0.5


def _local_body(r_ref, q_hbm, k_hbm, v_hbm, o_ref, m_ref, l_ref,
                qbuf, kbuf, vbuf, qsem, ksem, vsem):
    r = r_ref[0]
    h = pl.program_id(0)
    slot = h % 2

    def make_dmas(hh, sl):
        q_dma = pltpu.make_async_copy(
            q_hbm.at[r, :, hh, :], qbuf.at[sl], qsem.at[sl])
        k_dma = pltpu.make_async_copy(
            k_hbm.at[r, :, hh, :], kbuf.at[sl], ksem.at[sl])
        v_dma = pltpu.make_async_copy(
            v_hbm.at[r, :, hh, :], vbuf.at[sl], vsem.at[sl])
        return q_dma, k_dma, v_dma

    @pl.when(h == 0)
    def _():
        for dma in make_dmas(h, slot):
            dma.start()

    @pl.when(h + 1 < H)
    def _():
        for dma in make_dmas(h + 1, 1 - slot):
            dma.start()

    for dma in make_dmas(h, slot):
        dma.wait()

    q = qbuf[slot].astype(jnp.bfloat16)
    k = kbuf[slot].astype(jnp.bfloat16)
    s = lax.dot_general(
        q, k, (((1,), (1,)), ((), ())),
        preferred_element_type=jnp.float32) * SCALE
    m = jnp.max(s, axis=1, keepdims=True)
    p = jnp.exp(s - m)
    l = jnp.sum(p, axis=1, keepdims=True)
    v = vbuf[slot].astype(jnp.bfloat16)
    o = lax.dot_general(
        p.astype(jnp.bfloat16), v, (((1,), (0,)), ((), ())),
        preferred_element_type=jnp.float32)
    o_ref[pl.ds(h, 1)] = o[None]
    m_ref[pl.ds(h, 1)] = m[None]
    l_ref[pl.ds(h, 1)] = l[None]


def _local_partial(Q, K, V, rid):
    grid_spec = pltpu.PrefetchScalarGridSpec(
        num_scalar_prefetch=1,
        grid=(H,),
        in_specs=[
            pl.BlockSpec(memory_space=pltpu.MemorySpace.HBM),
            pl.BlockSpec(memory_space=pltpu.MemorySpace.HBM),
            pl.BlockSpec(memory_space=pltpu.MemorySpace.HBM),
        ],
        out_specs=[
            pl.BlockSpec((H, SQ, D), lambda i, r: (0, 0, 0)),
            pl.BlockSpec((H, SQ, 1), lambda i, r: (0, 0, 0)),
            pl.BlockSpec((H, SQ, 1), lambda i, r: (0, 0, 0)),
        ],
        scratch_shapes=[
            pltpu.VMEM((2, SQ, D), jnp.float32),
            pltpu.VMEM((2, SKV, D), jnp.float32),
            pltpu.VMEM((2, SKV, D), jnp.float32),
            pltpu.SemaphoreType.DMA((2,)),
            pltpu.SemaphoreType.DMA((2,)),
            pltpu.SemaphoreType.DMA((2,)),
        ],
    )
    return pl.pallas_call(
        _local_body,
        grid_spec=grid_spec,
        out_shape=[
            jax.ShapeDtypeStruct((H, SQ, D), jnp.float32),
            jax.ShapeDtypeStruct((H, SQ, 1), jnp.float32),
            jax.ShapeDtypeStruct((H, SQ, 1), jnp.float32),
        ],
    )(rid, Q, K, V)


def _combine_body(o_ref, m_ref, l_ref, out_ref,
                  acc_o, ml_acc, zrecv_o, zrecv_ml,
                  z_send_o_sem, z_recv_o_sem, z_send_ml_sem, z_recv_ml_sem,
                  xy_send_sem, xy_recv_sem):
    mx = lax.axis_index("x")
    my = lax.axis_index("y")
    mz = lax.axis_index("z")
    rid = mx * 4 + my

    partners = [
        (mx, my, mz ^ 1),
        (mx, my, mz ^ 2),
        (mx, my ^ 1, mz),
        (mx, my ^ 2, mz),
        (mx ^ 1, my, mz),
    ]
    barrier = pltpu.get_barrier_semaphore()
    for dev in partners:
        pl.semaphore_signal(barrier, inc=1, device_id=dev,
                            device_id_type=pl.DeviceIdType.MESH)
    pl.semaphore_wait(barrier, len(partners))

    acc_o[...] = o_ref[...]
    m_acc = m_ref[...]
    l_acc = l_ref[...]

    for s in range(2):
        ml_acc[0] = m_acc
        ml_acc[1] = l_acc
        dev = (mx, my, mz ^ (1 << s))
        rdma_o = pltpu.make_async_remote_copy(
            src_ref=acc_o, dst_ref=zrecv_o.at[s],
            send_sem=z_send_o_sem.at[s], recv_sem=z_recv_o_sem.at[s],
            device_id=dev, device_id_type=pl.DeviceIdType.MESH)
        rdma_ml = pltpu.make_async_remote_copy(
            src_ref=ml_acc, dst_ref=zrecv_ml.at[s],
            send_sem=z_send_ml_sem.at[s], recv_sem=z_recv_ml_sem.at[s],
            device_id=dev, device_id_type=pl.DeviceIdType.MESH)
        rdma_o.start()
        rdma_ml.start()
        rdma_ml.wait()
        rdma_o.wait()

        m_other = zrecv_ml[s, 0]
        l_other = zrecv_ml[s, 1]
        m_new = jnp.maximum(m_acc, m_other)
        a_self = jnp.exp(m_acc - m_new)
        a_other = jnp.exp(m_other - m_new)
        acc_o[...] = acc_o[...] * a_self + zrecv_o[s] * a_other
        l_acc = l_acc * a_self + l_other * a_other
        m_acc = m_new

    final = acc_o[...] / l_acc
    out_ref[pl.ds(rid, 1)] = jnp.swapaxes(final, 0, 1)[None]

    xy_partners = [
        (mx, my ^ 1, mz),
        (mx, my ^ 2, mz),
        (mx ^ 1, my, mz),
    ]
    for k in range(3):
        size = 1 << k
        base = (rid // size) * size
        rdma = pltpu.make_async_remote_copy(
            src_ref=out_ref.at[pl.ds(base, size)],
            dst_ref=out_ref.at[pl.ds(base, size)],
            send_sem=xy_send_sem.at[k], recv_sem=xy_recv_sem.at[k],
            device_id=xy_partners[k], device_id_type=pl.DeviceIdType.MESH)
        rdma.start()
        rdma.wait()


def _combine(o_part, m_part, l_part):
    return pl.pallas_call(
        _combine_body,
        in_specs=[
            pl.BlockSpec(memory_space=pltpu.VMEM),
            pl.BlockSpec(memory_space=pltpu.VMEM),
            pl.BlockSpec(memory_space=pltpu.VMEM),
        ],
        out_specs=pl.BlockSpec(memory_space=pltpu.VMEM),
        out_shape=jax.ShapeDtypeStruct((B, SQ, H, D), jnp.float32),
        scratch_shapes=[
            pltpu.VMEM((H, SQ, D), jnp.float32),
            pltpu.VMEM((2, H, SQ, 1), jnp.float32),
            pltpu.VMEM((2, H, SQ, D), jnp.float32),
            pltpu.VMEM((2, 2, H, SQ, 1), jnp.float32),
            pltpu.SemaphoreType.DMA((2,)),
            pltpu.SemaphoreType.DMA((2,)),
            pltpu.SemaphoreType.DMA((2,)),
            pltpu.SemaphoreType.DMA((2,)),
            pltpu.SemaphoreType.DMA((3,)),
            pltpu.SemaphoreType.DMA((3,)),
        ],
        compiler_params=pltpu.CompilerParams(collective_id=0),
    )(o_part, m_part, l_part)


def kernel(Q, K, V):
    rid = (lax.axis_index("x") * 4 + lax.axis_index("y")).astype(jnp.int32)
    rid_arr = jnp.reshape(rid, (1,))
    o_part, m_part, l_part = _local_partial(Q, K, V, rid_arr)
    return _combine(o_part, m_part, l_part)


# device time: 39383 ns/iter; 4.6001x vs baseline; 1.1235x over previous
import jax
import jax.numpy as jnp
from jax import lax
from jax.experimental import pallas as pl
from jax.experimental.pallas import tpu as pltpu

B, SQ, H, D = 8, 8, 16, 128
SKV = 1024
SCALE = D ** -0.5


def _local_body(r_ref, q_hbm, k_hbm, v_hbm, o_ref, m_ref, l_ref,
                qbuf, kbuf, vbuf, qsem, ksem, vsem):
    r = r_ref[0]
    h = pl.program_id(0)
    slot = h % 2

    def make_dmas(hh, sl):
        q_dma = pltpu.make_async_copy(
            q_hbm.at[r, :, hh, :], qbuf.at[sl], qsem.at[sl])
        k_dma = pltpu.make_async_copy(
            k_hbm.at[r, :, hh, :], kbuf.at[sl], ksem.at[sl])
        v_dma = pltpu.make_async_copy(
            v_hbm.at[r, :, hh, :], vbuf.at[sl], vsem.at[sl])
        return q_dma, k_dma, v_dma

    @pl.when(h == 0)
    def _():
        for dma in make_dmas(h, slot):
            dma.start()

    @pl.when(h + 1 < H)
    def _():
        for dma in make_dmas(h + 1, 1 - slot):
            dma.start()

    for dma in make_dmas(h, slot):
        dma.wait()

    q = qbuf[slot].astype(jnp.bfloat16)
    k = kbuf[slot].astype(jnp.bfloat16)
    s = lax.dot_general(
        q, k, (((1,), (1,)), ((), ())),
        preferred_element_type=jnp.float32) * SCALE
    m = jnp.max(s, axis=1, keepdims=True)
    p = jnp.exp(s - m)
    l = jnp.sum(p, axis=1, keepdims=True)
    v = vbuf[slot].astype(jnp.bfloat16)
    o = lax.dot_general(
        p.astype(jnp.bfloat16), v, (((1,), (0,)), ((), ())),
        preferred_element_type=jnp.float32)
    o_ref[pl.ds(h, 1)] = o[None]
    m_ref[pl.ds(h, 1)] = m[None]
    l_ref[pl.ds(h, 1)] = l[None]


def _local_partial(Q, K, V, rid):
    grid_spec = pltpu.PrefetchScalarGridSpec(
        num_scalar_prefetch=1,
        grid=(H,),
        in_specs=[
            pl.BlockSpec(memory_space=pltpu.MemorySpace.HBM),
            pl.BlockSpec(memory_space=pltpu.MemorySpace.HBM),
            pl.BlockSpec(memory_space=pltpu.MemorySpace.HBM),
        ],
        out_specs=[
            pl.BlockSpec((H, SQ, D), lambda i, r: (0, 0, 0)),
            pl.BlockSpec((H, SQ, 1), lambda i, r: (0, 0, 0)),
            pl.BlockSpec((H, SQ, 1), lambda i, r: (0, 0, 0)),
        ],
        scratch_shapes=[
            pltpu.VMEM((2, SQ, D), jnp.float32),
            pltpu.VMEM((2, SKV, D), jnp.float32),
            pltpu.VMEM((2, SKV, D), jnp.float32),
            pltpu.SemaphoreType.DMA((2,)),
            pltpu.SemaphoreType.DMA((2,)),
            pltpu.SemaphoreType.DMA((2,)),
        ],
    )
    return pl.pallas_call(
        _local_body,
        grid_spec=grid_spec,
        out_shape=[
            jax.ShapeDtypeStruct((H, SQ, D), jnp.float32),
            jax.ShapeDtypeStruct((H, SQ, 1), jnp.float32),
            jax.ShapeDtypeStruct((H, SQ, 1), jnp.float32),
        ],
    )(rid, Q, K, V)


def _combine_body(o_ref, m_ref, l_ref, out_ref,
                  acc_o, ml_acc, zsend_o, zrecv_o, zrecv_ml, gbuf,
                  z_send_o_sem, z_recv_o_sem, z_send_ml_sem, z_recv_ml_sem,
                  xy_send_sem, xy_recv_sem):
    mx = lax.axis_index("x")
    my = lax.axis_index("y")
    mz = lax.axis_index("z")
    rid = mx * 4 + my

    partners = [
        (mx, my, mz ^ 1),
        (mx, my, mz ^ 2),
        (mx, my ^ 1, mz),
        (mx, my ^ 2, mz),
        (mx ^ 1, my, mz),
    ]
    barrier = pltpu.get_barrier_semaphore()
    for dev in partners:
        pl.semaphore_signal(barrier, inc=1, device_id=dev,
                            device_id_type=pl.DeviceIdType.MESH)
    pl.semaphore_wait(barrier, len(partners))

    acc_o[...] = o_ref[...]
    m_acc = m_ref[...]
    l_acc = l_ref[...]

    for s in range(2):
        ml_acc[0] = m_acc
        ml_acc[1] = l_acc
        zsend_o[s] = acc_o[...].astype(jnp.bfloat16)
        dev = (mx, my, mz ^ (1 << s))
        rdma_o = pltpu.make_async_remote_copy(
            src_ref=zsend_o.at[s], dst_ref=zrecv_o.at[s],
            send_sem=z_send_o_sem.at[s], recv_sem=z_recv_o_sem.at[s],
            device_id=dev, device_id_type=pl.DeviceIdType.MESH)
        rdma_ml = pltpu.make_async_remote_copy(
            src_ref=ml_acc, dst_ref=zrecv_ml.at[s],
            send_sem=z_send_ml_sem.at[s], recv_sem=z_recv_ml_sem.at[s],
            device_id=dev, device_id_type=pl.DeviceIdType.MESH)
        rdma_o.start()
        rdma_ml.start()
        rdma_ml.wait()
        rdma_o.wait()

        m_other = zrecv_ml[s, 0]
        l_other = zrecv_ml[s, 1]
        m_new = jnp.maximum(m_acc, m_other)
        a_self = jnp.exp(m_acc - m_new)
        a_other = jnp.exp(m_other - m_new)
        acc_o[...] = (acc_o[...] * a_self
                      + zrecv_o[s].astype(jnp.float32) * a_other)
        l_acc = l_acc * a_self + l_other * a_other
        m_acc = m_new

    final = acc_o[...] / l_acc
    gbuf[pl.ds(rid, 1)] = jnp.swapaxes(final, 0, 1)[None].astype(jnp.bfloat16)

    xy_partners = [
        (mx, my ^ 1, mz),
        (mx, my ^ 2, mz),
        (mx ^ 1, my, mz),
    ]
    for k in range(3):
        size = 1 << k
        base = (rid // size) * size
        rdma = pltpu.make_async_remote_copy(
            src_ref=gbuf.at[pl.ds(base, size)],
            dst_ref=gbuf.at[pl.ds(base, size)],
            send_sem=xy_send_sem.at[k], recv_sem=xy_recv_sem.at[k],
            device_id=xy_partners[k], device_id_type=pl.DeviceIdType.MESH)
        rdma.start()
        rdma.wait()

    out_ref[...] = gbuf[...].astype(jnp.float32)


def _combine(o_part, m_part, l_part):
    return pl.pallas_call(
        _combine_body,
        in_specs=[
            pl.BlockSpec(memory_space=pltpu.VMEM),
            pl.BlockSpec(memory_space=pltpu.VMEM),
            pl.BlockSpec(memory_space=pltpu.VMEM),
        ],
        out_specs=pl.BlockSpec(memory_space=pltpu.VMEM),
        out_shape=jax.ShapeDtypeStruct((B, SQ, H, D), jnp.float32),
        scratch_shapes=[
            pltpu.VMEM((H, SQ, D), jnp.float32),
            pltpu.VMEM((2, H, SQ, 1), jnp.float32),
            pltpu.VMEM((2, H, SQ, D), jnp.bfloat16),
            pltpu.VMEM((2, H, SQ, D), jnp.bfloat16),
            pltpu.VMEM((2, 2, H, SQ, 1), jnp.float32),
            pltpu.VMEM((B, SQ, H, D), jnp.bfloat16),
            pltpu.SemaphoreType.DMA((2,)),
            pltpu.SemaphoreType.DMA((2,)),
            pltpu.SemaphoreType.DMA((2,)),
            pltpu.SemaphoreType.DMA((2,)),
            pltpu.SemaphoreType.DMA((3,)),
            pltpu.SemaphoreType.DMA((3,)),
        ],
        compiler_params=pltpu.CompilerParams(collective_id=0),
    )(o_part, m_part, l_part)


def kernel(Q, K, V):
    rid = (lax.axis_index("x") * 4 + lax.axis_index("y")).astype(jnp.int32)
    rid_arr = jnp.reshape(rid, (1,))
    o_part, m_part, l_part = _local_partial(Q, K, V, rid_arr)
    return _combine(o_part, m_part, l_part)
